# baseline (speedup 1.0000x reference)
import functools

import jax
import jax.numpy as jnp
from jax import lax
from jax.experimental import pallas as pl
from jax.experimental.pallas import tpu as pltpu
from jax.experimental.pallas import tpu_sc as plsc


_MESH = plsc.VectorSubcoreMesh(
    core_axis_name="c", subcore_axis_name="s", num_cores=1
)


@functools.partial(
    pl.kernel,
    mesh=_MESH,
    out_type=jax.ShapeDtypeStruct((1,), jnp.float32),
)
def _scalar_gather(x_hbm, out_hbm):
    sid = lax.axis_index("s")

    @pl.when(sid == 0)
    def _():
        pltpu.sync_copy(x_hbm, out_hbm)


def kernel(x):
    return _scalar_gather(x.reshape(1))[0].reshape(())


# speedup vs baseline: 15.2098x; 15.2098x over previous
import jax
import jax.numpy as jnp
from jax.experimental import pallas as pl


def _copy_body(x_ref, o_ref):
    o_ref[...] = x_ref[...]


def kernel(x):
    xr = x.reshape(1, 1)
    out = pl.pallas_call(
        _copy_body,
        out_shape=jax.ShapeDtypeStruct((1, 1), jnp.float32),
    )(xr)
    return out[0, 0]
